# baseline (speedup 1.0000x reference)
import math

import jax
import jax.numpy as jnp
from jax import lax
from jax.experimental import pallas as pl
from jax.experimental.pallas import tpu as pltpu
from jax.experimental.pallas import tpu_sc as plsc

NE = 57600
HW = 576
CK = 64
CV = 512
CO = 1024
CELL = 128
NCELL = NE // CELL
MPAD = 512
TOPK = 20
NEG = -3.0e38

NC, NS = 2, 16
NW = NC * NS
ROWS_W = HW // NW
MT_ROWS = HW + 16

NEB = 2304
NCB = NEB // CELL


def _scores_body(qk_ref, mk_ref, mv_ref, s_ref, m_ref, mvt_ref):
    mk = mk_ref[...]
    qk = qk_ref[...]
    ab = lax.dot_general(
        qk.astype(jnp.bfloat16), mk.astype(jnp.bfloat16),
        (((0,), (0,)), ((), ())),
        preferred_element_type=jnp.float32)
    asq = jnp.sum(mk * mk, axis=0)
    s = (2.0 * ab - asq[None, :]) * (1.0 / math.sqrt(CK))
    for c in range(NCB):
        s_ref[c] = s[:, c * CELL:(c + 1) * CELL]
    st = s.T
    m_ref[0] = jnp.max(st.reshape(NCB, CELL, HW), axis=1)
    mvt = mv_ref[...].T
    rnd = mvt.astype(jnp.bfloat16).astype(jnp.float32)
    bits = lax.bitcast_convert_type(rnd, jnp.int32)
    lo = lax.shift_right_logical(bits[:, :CV // 2], 16)
    hi = bits[:, CV // 2:]
    mvt_ref[...] = lo | hi


def _scores(qk2, mk2, mv2):
    return pl.pallas_call(
        _scores_body,
        grid=(NE // NEB,),
        in_specs=[
            pl.BlockSpec((CK, HW), lambda i: (0, 0)),
            pl.BlockSpec((CK, NEB), lambda i: (0, i)),
            pl.BlockSpec((CV, NEB), lambda i: (0, i)),
        ],
        out_specs=[
            pl.BlockSpec((NCB, HW, CELL), lambda i: (i, 0, 0)),
            pl.BlockSpec((1, NCB, HW), lambda i: (i, 0, 0)),
            pl.BlockSpec((NEB, CV // 2), lambda i: (i, 0)),
        ],
        out_shape=[
            jax.ShapeDtypeStruct((NCELL, HW, CELL), jnp.float32),
            jax.ShapeDtypeStruct((NE // NEB, NCB, HW), jnp.float32),
            jax.ShapeDtypeStruct((NE, CV // 2), jnp.int32),
        ],
    )(qk2, mk2, mv2)


def _iota16():
    return lax.iota(jnp.int32, 16)


def _rev(x):
    return lax.rev(x, (0,))


def _splat_max(x):
    return plsc.cummax(_rev(plsc.cummax(x)))


def _splat_sum_nonneg(x):
    return plsc.cummax(_rev(plsc.cumsum(x)))


def _round_bf16(x):
    b = plsc.bitcast(x, jnp.int32)
    lsb = lax.shift_right_logical(b, 16) & 1
    b = b + 0x7FFF + lsb
    b = b & jnp.int32(-65536)
    return plsc.bitcast(b, jnp.float32)


def _select_top(loads, scatter_negs, gathers, nvec, emits):
    n = len(loads)
    it16 = _iota16()
    lane0 = it16 == 0
    negv = jnp.full((16,), NEG, jnp.float32)
    zerov = jnp.zeros((16,), jnp.int32)
    ngrp = (nvec + 15) // 16

    def init_body(v, c):
        out = []
        for i in range(n):
            cm, cv = c[2 * i], c[2 * i + 1]
            x = loads[i](v)
            upd = x > cm
            out += [jnp.where(upd, x, cm), jnp.where(upd, v, cv)]
        return tuple(out)

    st = lax.fori_loop(0, nvec, init_body, (negv, zerov) * n, unroll=4)
    colmax = [st[2 * i] for i in range(n)]
    colv = [st[2 * i + 1] for i in range(n)]

    for j in range(TOPK):
        g = [_splat_max(colmax[i]) for i in range(n)]
        p = [_splat_max(jnp.where(colmax[i] == g[i], colv[i] * 16 + it16,
                                  jnp.int32(-1))) for i in range(n)]
        lane = [p[i] & 15 for i in range(n)]
        for i in range(n):
            emits[i](j, g[i], p[i])
            scatter_negs[i](p[i], negv, lane0)
        newm = [negv] * n
        newv = [zerov] * n
        for gi in range(ngrp):
            vid = it16 + gi * 16
            base = jnp.minimum(vid, nvec - 1) * 16
            for i in range(n):
                vals = jnp.where(vid < nvec,
                                 gathers[i](base + lane[i]), NEG)
                upd = vals > newm[i]
                newm[i] = jnp.where(upd, vals, newm[i])
                newv[i] = jnp.where(upd, vid, newv[i])
        for i in range(n):
            colh = _splat_max(newm[i])
            vnew = _splat_max(
                jnp.where(newm[i] == colh, newv[i], jnp.int32(-1)))
            colmax[i] = jnp.where(it16 == lane[i], colh, colmax[i])
            colv[i] = jnp.where(it16 == lane[i], vnew, colv[i])


def _fused_body(s3, mrows, mvt, keep, memt,
                m_all_v, cand_v, rows_v, out_s, keep_v, sem):
    wid = lax.axis_index("s") * NC + lax.axis_index("c")
    it16 = _iota16()
    pltpu.sync_copy(keep, keep_v)
    keep0 = keep_v[pl.ds(0, 16)]
    keep1 = keep_v[pl.ds(16, 16)]

    base_row = wid * ROWS_W
    mids0 = base_row + it16
    mids1 = (base_row + 16 + it16) % HW
    pltpu.async_copy(mrows.at[mids0], m_all_v.at[pl.ds(0, 16)], sem).wait()
    pltpu.async_copy(mrows.at[mids1], m_all_v.at[pl.ds(16, 16)], sem).wait()

    def row_body(rr, _):
        rs = [rr * 2, rr * 2 + 1]
        rows = [base_row + r for r in rs]
        rvecs = [jnp.zeros((16,), jnp.int32) + r for r in rs]

        cells = [[jnp.zeros((16,), jnp.int32), jnp.zeros((16,), jnp.int32)]
                 for _i in range(2)]

        def mk_emit_cell(i):
            def emit_cell(j, g, p):
                cells[i][j // 16] = jnp.where(
                    it16 == (j % 16), p, cells[i][j // 16])
            return emit_cell

        _select_top(
            [lambda v, r=r: m_all_v[r, pl.ds(v * 16, 16)] for r in rs],
            [lambda p, x, m, rv=rv: plsc.store_scatter(
                m_all_v, [rv, p], x, mask=m) for rv in rvecs],
            [lambda ids, rv=rv: plsc.load_gather(m_all_v, [rv, ids])
             for rv in rvecs],
            MPAD // 16, [mk_emit_cell(0), mk_emit_cell(1)])

        copies = []
        for i in range(2):
            gids0 = cells[i][0] * HW + rows[i]
            gids1 = jnp.where(it16 + 16 < TOPK, cells[i][1],
                              100 + it16) * HW + rows[i]
            copies.append(pltpu.async_copy(
                s3.at[gids0], cand_v.at[pl.ds(32 * i, 16)], sem))
            copies.append(pltpu.async_copy(
                s3.at[gids1], cand_v.at[pl.ds(32 * i + 16, 16)], sem))
        for d in copies:
            d.wait()

        vals = [[jnp.full((16,), NEG, jnp.float32),
                 jnp.full((16,), NEG, jnp.float32)] for _i in range(2)]
        idxs = [[jnp.zeros((16,), jnp.int32), jnp.zeros((16,), jnp.int32)]
                for _i in range(2)]

        def mk_emit_cand(i):
            def emit_cand(j, g, p):
                q = p // CELL
                off = p - q * CELL
                cq = _splat_max(jnp.maximum(
                    jnp.where(it16 == q, cells[i][0], jnp.int32(-1)),
                    jnp.where(it16 + 16 == q, cells[i][1], jnp.int32(-1))))
                n = cq * CELL + off
                vals[i][j // 16] = jnp.where(
                    it16 == (j % 16), g, vals[i][j // 16])
                idxs[i][j // 16] = jnp.where(
                    it16 == (j % 16), n, idxs[i][j // 16])
            return emit_cand

        _select_top(
            [lambda v, b=32 * 0: cand_v[b + v // 8, pl.ds((v % 8) * 16, 16)],
             lambda v, b=32 * 1: cand_v[b + v // 8, pl.ds((v % 8) * 16, 16)]],
            [lambda p, x, m, b=32 * 0: plsc.store_scatter(
                cand_v, [b + p // CELL, p % CELL], x, mask=m),
             lambda p, x, m, b=32 * 1: plsc.store_scatter(
                cand_v, [b + p // CELL, p % CELL], x, mask=m)],
            [lambda ids, b=32 * 0: plsc.load_gather(
                cand_v, [b + ids // CELL, ids % CELL]),
             lambda ids, b=32 * 1: plsc.load_gather(
                cand_v, [b + ids // CELL, ids % CELL])],
            TOPK * CELL // 16, [mk_emit_cand(0), mk_emit_cand(1)])

        ws = []
        for i in range(2):
            vmax = _splat_max(vals[i][0])
            e0 = jnp.exp(vals[i][0] - vmax) * keep0
            e1 = jnp.exp(vals[i][1] - vmax) * keep1
            inv = 1.0 / _splat_sum_nonneg(e0 + e1)
            ws.append((_round_bf16(e0 * inv), _round_bf16(e1 * inv)))

        copies = []
        for i in range(2):
            copies.append(pltpu.async_copy(
                mvt.at[idxs[i][0]], rows_v.at[pl.ds(32 * i, 16)], sem))
            copies.append(pltpu.async_copy(
                mvt.at[jnp.where(it16 + 16 < TOPK, idxs[i][1],
                                 wid * 16 + it16)],
                rows_v.at[pl.ds(32 * i + 16, 16)], sem))
        for d in copies:
            d.wait()
        for t in range(CV // 32):
            for i in range(2):
                acc_lo = jnp.zeros((16,), jnp.float32)
                acc_hi = jnp.zeros((16,), jnp.float32)
                for j in range(TOPK):
                    wj = ws[i][j // 16][j % 16]
                    bits = rows_v[32 * i + j, pl.ds(t * 16, 16)]
                    lo = plsc.bitcast(lax.shift_left(bits, 16), jnp.float32)
                    hi = plsc.bitcast(bits & jnp.int32(-65536), jnp.float32)
                    acc_lo = acc_lo + wj * lo
                    acc_hi = acc_hi + wj * hi
                out_s[rs[i], pl.ds(t * 16, 16)] = acc_lo
                out_s[rs[i], pl.ds(CV // 2 + t * 16, 16)] = acc_hi
        return 0

    lax.fori_loop(0, ROWS_W // 2, row_body, 0)

    oids0 = base_row + it16
    oids1 = jnp.where(it16 < ROWS_W - 16, base_row + 16 + it16, HW + it16)
    pltpu.async_copy(out_s.at[pl.ds(0, 16)], memt.at[oids0], sem).wait()
    pltpu.async_copy(out_s.at[pl.ds(16, 16)], memt.at[oids1], sem).wait()


def _fused_sc(s3, mrows, mvt, keep):
    mesh = plsc.VectorSubcoreMesh(core_axis_name="c", subcore_axis_name="s",
                                  num_cores=NC, num_subcores=NS)
    fn = pl.kernel(
        _fused_body,
        out_type=jax.ShapeDtypeStruct((MT_ROWS, CV), jnp.float32),
        mesh=mesh,
        compiler_params=pltpu.CompilerParams(needs_layout_passes=False),
        scratch_types=[
            pltpu.VMEM((32, MPAD), jnp.float32),
            pltpu.VMEM((64, CELL), jnp.float32),
            pltpu.VMEM((64, CV // 2), jnp.int32),
            pltpu.VMEM((32, CV), jnp.float32),
            pltpu.VMEM((32,), jnp.float32),
            pltpu.SemaphoreType.DMA,
        ],
    )
    s3flat = s3.reshape(NCELL * HW, CELL)
    return fn(s3flat, mrows, mvt, keep)


def _epilogue_body(mem_ref, qv_ref, lnw_ref, lnb_ref, w2_ref, b_ref, out_ref):
    x = qv_ref[...] + mem_ref[...]
    mu = jnp.mean(x, axis=1, keepdims=True)
    var = jnp.mean((x - mu) * (x - mu), axis=1, keepdims=True)
    nrm = (x - mu) * lax.rsqrt(var + 1e-5) * lnw_ref[...] + lnb_ref[...]
    out_ref[...] = lax.dot_general(
        w2_ref[...].astype(jnp.bfloat16), nrm.astype(jnp.bfloat16),
        (((1,), (1,)), ((), ())),
        preferred_element_type=jnp.float32) + b_ref[...]


def _epilogue(memt, qvt, lnw, lnb, w2, b2):
    return pl.pallas_call(
        _epilogue_body,
        out_shape=jax.ShapeDtypeStruct((CO, HW), jnp.float32),
    )(memt, qvt, lnw, lnb, w2, b2)


def kernel(mk, qk, mv, qv, ln_w, ln_b, conv_w, conv_b, k):
    mk2 = mk[0]
    qk2 = qk[0]
    mv2 = mv[0]
    s3, m3, mvt = _scores(qk2, mk2, mv2)
    m = jnp.concatenate(
        [m3.reshape(NCELL, HW).T,
         jnp.full((HW, MPAD - NCELL), NEG, jnp.float32)], axis=1)
    keep = (jnp.arange(32) < jnp.minimum(k, TOPK)).astype(jnp.float32)
    memt = _fused_sc(s3, m, mvt, keep)
    qvt = qv.reshape(CV, HW).T
    out = _epilogue(memt[:HW], qvt, ln_w.reshape(1, CV),
                    ln_b.reshape(1, CV), conv_w[:, :, 0, 0],
                    conv_b.reshape(CO, 1))
    return out.reshape(1, CO, 24, 24)
